# baseline (speedup 1.0000x reference)
import jax
import jax.numpy as jnp
from jax.experimental import pallas as pl

_USER = 60000
_ITEM = 40000
_N = _USER + _ITEM
_LAT = 32
_RIS = 0.2


def _spmm(indices, values, mat):
    gathered = jnp.take(mat, indices[1], axis=0) * values[:, None]
    return jax.ops.segment_sum(gathered, indices[0], num_segments=_N)


def _l2norm(x):
    n = jnp.sqrt(jnp.sum(x * x, axis=1, keepdims=True))
    return x / jnp.maximum(n, 1e-12)


def _sum3_kernel(a_ref, b_ref, c_ref, o_ref):
    o_ref[...] = a_ref[...] + b_ref[...] + c_ref[...]


def _sum3(a, b, c):
    bm = 1000
    return pl.pallas_call(
        _sum3_kernel,
        grid=(_N // bm,),
        in_specs=[pl.BlockSpec((bm, _LAT), lambda i: (i, 0))] * 3,
        out_specs=pl.BlockSpec((bm, _LAT), lambda i: (i, 0)),
        out_shape=jax.ShapeDtypeStruct((_N, _LAT), jnp.float32),
    )(a, b, c)


def kernel(adj_indices, adj_values, image_adj_indices, image_adj_values,
           text_adj_indices, text_adj_values, image_embedding, text_embedding,
           uEmbeds, iEmbeds, image_trans, text_trans):
    lrelu = lambda x: jax.nn.leaky_relu(x, negative_slope=0.2)
    image_feats = lrelu(image_embedding @ image_trans)
    text_feats = lrelu(text_embedding @ text_trans)
    base = jnp.concatenate([uEmbeds, iEmbeds], axis=0)

    embedsImageAdj = _spmm(image_adj_indices, image_adj_values, base)
    embedsImage = _spmm(adj_indices, adj_values,
                        jnp.concatenate([uEmbeds, _l2norm(image_feats)], axis=0))
    embedsImage_ = _spmm(adj_indices, adj_values,
                         jnp.concatenate([embedsImage[:_USER], iEmbeds], axis=0))
    z0 = embedsImage + embedsImage_ + _RIS * embedsImageAdj
    z1 = _spmm(adj_indices, adj_values, z0)
    z2 = _spmm(adj_indices, adj_values, z1)
    embeds_visual = _sum3(z0, z1, z2)

    embedsTextAdj = _spmm(text_adj_indices, text_adj_values, base)
    embedsText = _spmm(adj_indices, adj_values,
                       jnp.concatenate([uEmbeds, _l2norm(text_feats)], axis=0))
    embedsText_ = _spmm(adj_indices, adj_values,
                        jnp.concatenate([embedsText[:_USER], iEmbeds], axis=0))
    t0 = embedsText + embedsText_ + _RIS * embedsTextAdj
    t1 = _spmm(adj_indices, adj_values, t0)
    t2 = _spmm(adj_indices, adj_values, t1)
    embeds_text = _sum3(t0, t1, t2)

    embeds = jnp.concatenate([embeds_visual, embeds_text], axis=-1)
    return embeds[:_USER], embeds[_USER:]


# speedup vs baseline: 3.4877x; 3.4877x over previous
import functools

import jax
import jax.numpy as jnp
from jax import lax
from jax.experimental import pallas as pl
from jax.experimental.pallas import tpu as pltpu
from jax.experimental.pallas import tpu_sc as plsc

USER_N = 60000
ITEM_N = 40000
NODES = USER_N + ITEM_N
LAT = 32
EDGES = 1600000
RIS = 0.2

NC = 2
NS = 16
HALF = NODES // NC
DUMP = 1280
ACC_ROWS = HALF + DUMP
ROWS_MAIN = 3128
ROWS_LAST = HALF - 15 * ROWS_MAIN

EPAD = 1638400
ES = EPAD // NS
BATCH = 1024
SUB = 128
NBATCH = ES // BATCH
NSUB = BATCH // SUB


def _spmm_body(dst_hbm, src_hbm, val_hbm, table_hbm, init_hbm, out_hbm,
               acc, dstb, srcb, valb, idxb, src_sc, idx_sc, rows, sem):
    c = lax.axis_index("c")
    s = lax.axis_index("s")
    lo = c * HALF
    r0 = s * ROWS_MAIN

    @pl.when(s < NS - 1)
    def _():
        pltpu.sync_copy(init_hbm.at[pl.ds(lo + r0, ROWS_MAIN)],
                        acc.at[pl.ds(r0, ROWS_MAIN)])

    @pl.when(s == NS - 1)
    def _():
        pltpu.sync_copy(init_hbm.at[pl.ds(lo + r0, ROWS_LAST)],
                        acc.at[pl.ds(r0, ROWS_LAST)])

    plsc.subcore_barrier()

    iota = lax.iota(jnp.int32, 16)
    e_base = s * ES

    def batch_body(b, _):
        e0 = e_base + b * BATCH
        pltpu.sync_copy(dst_hbm.at[pl.ds(e0, BATCH)], dstb)
        pltpu.sync_copy(src_hbm.at[pl.ds(e0, BATCH)], srcb)
        pltpu.sync_copy(val_hbm.at[pl.ds(e0, BATCH)], valb)

        def filt(j, _):
            d16 = dstb[pl.ds(j * 16, 16)]
            rl = d16 - lo
            mask = (rl >= 0) & (rl < HALF)
            dump = HALF + (j % (DUMP // 16)) * 16 + iota
            idxb[pl.ds(j * 16, 16)] = jnp.where(mask, rl, dump)
            v16 = valb[pl.ds(j * 16, 16)]
            valb[pl.ds(j * 16, 16)] = jnp.where(mask, v16, 0.0)
            return 0

        lax.fori_loop(0, BATCH // 16, filt, 0, unroll=4)

        def sub_body(k, _):
            o = k * SUB
            def stage(t, _):
                src_sc[pl.ds(t * 16, 16)] = srcb[pl.ds(o + t * 16, 16)]
                idx_sc[pl.ds(t * 16, 16)] = idxb[pl.ds(o + t * 16, 16)]
                return 0
            lax.fori_loop(0, SUB // 16, stage, 0, unroll=8)

            pltpu.async_copy(table_hbm.at[src_sc], rows, sem).wait()

            def mul(r, _):
                bv = plsc.load_gather(valb, [jnp.full((16,), o + r, jnp.int32)])
                rows[r, pl.ds(0, 16)] = rows[r, pl.ds(0, 16)] * bv
                rows[r, pl.ds(16, 16)] = rows[r, pl.ds(16, 16)] * bv
                return 0

            lax.fori_loop(0, SUB, mul, 0, unroll=4)

            pltpu.sync_copy(rows, acc.at[idx_sc], add=True)
            return 0

        lax.fori_loop(0, NSUB, sub_body, 0)
        return 0

    lax.fori_loop(0, NBATCH, batch_body, 0)
    plsc.subcore_barrier()

    @pl.when(s < NS - 1)
    def _():
        pltpu.sync_copy(acc.at[pl.ds(r0, ROWS_MAIN)],
                        out_hbm.at[pl.ds(lo + r0, ROWS_MAIN)])

    @pl.when(s == NS - 1)
    def _():
        pltpu.sync_copy(acc.at[pl.ds(r0, ROWS_LAST)],
                        out_hbm.at[pl.ds(lo + r0, ROWS_LAST)])


_spmm_call = pl.kernel(
    _spmm_body,
    out_type=jax.ShapeDtypeStruct((NODES, LAT), jnp.float32),
    mesh=plsc.VectorSubcoreMesh(core_axis_name="c", subcore_axis_name="s"),
    scratch_types=[
        pltpu.VMEM_SHARED((ACC_ROWS, LAT), jnp.float32),
        pltpu.VMEM((BATCH,), jnp.int32),
        pltpu.VMEM((BATCH,), jnp.int32),
        pltpu.VMEM((BATCH,), jnp.float32),
        pltpu.VMEM((BATCH,), jnp.int32),
        pltpu.VMEM((SUB,), jnp.int32),
        pltpu.VMEM((SUB,), jnp.int32),
        pltpu.VMEM((SUB, LAT), jnp.float32),
        pltpu.SemaphoreType.DMA,
    ],
    compiler_params=pltpu.CompilerParams(needs_layout_passes=False,
                                         use_tc_tiling_on_sc=False),
)


def _spmm(dst, src, val, table, init):
    return _spmm_call(dst, src, val, table, init)


def _feat_kernel(emb_ref, w_ref, o_ref):
    x = jnp.dot(emb_ref[...], w_ref[...], preferred_element_type=jnp.float32)
    y = jnp.where(x >= 0, x, 0.2 * x)
    n = jnp.sqrt(jnp.sum(y * y, axis=1, keepdims=True))
    o_ref[...] = y / jnp.maximum(n, 1e-12)


def _feat_norm(emb, w):
    bm = 400
    f = emb.shape[1]
    return pl.pallas_call(
        _feat_kernel,
        grid=(ITEM_N // bm,),
        in_specs=[pl.BlockSpec((bm, f), lambda i: (i, 0)),
                  pl.BlockSpec((f, LAT), lambda i: (0, 0))],
        out_specs=pl.BlockSpec((bm, LAT), lambda i: (i, 0)),
        out_shape=jax.ShapeDtypeStruct((ITEM_N, LAT), jnp.float32),
    )(emb, w)


def _axpy_kernel(scale, a_ref, b_ref, o_ref):
    o_ref[...] = a_ref[...] + scale * b_ref[...]


def _axpy(a, b, scale):
    bm = 1000
    return pl.pallas_call(
        functools.partial(_axpy_kernel, scale),
        grid=(NODES // bm,),
        in_specs=[pl.BlockSpec((bm, LAT), lambda i: (i, 0))] * 2,
        out_specs=pl.BlockSpec((bm, LAT), lambda i: (i, 0)),
        out_shape=jax.ShapeDtypeStruct((NODES, LAT), jnp.float32),
    )(a, b)


def kernel(adj_indices, adj_values, image_adj_indices, image_adj_values,
           text_adj_indices, text_adj_values, image_embedding, text_embedding,
           uEmbeds, iEmbeds, image_trans, text_trans):
    pad = EPAD - EDGES
    pad_idx = (jnp.arange(pad, dtype=jnp.int32) * 997) % NODES
    pad_val = jnp.zeros((pad,), jnp.float32)

    def split_edges(indices, values):
        dst = jnp.concatenate([indices[0], pad_idx])
        src = jnp.concatenate([indices[1], pad_idx])
        val = jnp.concatenate([values, pad_val])
        return dst, src, val

    a_dst, a_src, a_val = split_edges(adj_indices, adj_values)
    i_dst, i_src, i_val = split_edges(image_adj_indices, image_adj_values)
    t_dst, t_src, t_val = split_edges(text_adj_indices, text_adj_values)

    img_n = _feat_norm(image_embedding, image_trans)
    txt_n = _feat_norm(text_embedding, text_trans)
    base = jnp.concatenate([uEmbeds, iEmbeds], axis=0)
    x_img = jnp.concatenate([uEmbeds, img_n], axis=0)
    x_txt = jnp.concatenate([uEmbeds, txt_n], axis=0)
    zeros = jnp.zeros((NODES, LAT), jnp.float32)

    def branch(dstv, srcv, valv, x_in):
        a_m = _spmm(dstv, srcv, valv, base, zeros)
        y = _spmm(a_dst, a_src, a_val, x_in, zeros)
        p = _axpy(y, a_m, RIS)
        x2 = jnp.concatenate([y[:USER_N], iEmbeds], axis=0)
        z0 = _spmm(a_dst, a_src, a_val, x2, p)
        z1 = _spmm(a_dst, a_src, a_val, z0, zeros)
        q = _axpy(z0, z1, 1.0)
        return _spmm(a_dst, a_src, a_val, z1, q)

    f_img = branch(i_dst, i_src, i_val, x_img)
    f_txt = branch(t_dst, t_src, t_val, x_txt)

    embeds = jnp.concatenate([f_img, f_txt], axis=-1)
    return embeds[:USER_N], embeds[USER_N:]


# speedup vs baseline: 5.4665x; 1.5674x over previous
import functools

import jax
import jax.numpy as jnp
from jax import lax
from jax.experimental import pallas as pl
from jax.experimental.pallas import tpu as pltpu
from jax.experimental.pallas import tpu_sc as plsc

USER_N = 60000
ITEM_N = 40000
NODES = USER_N + ITEM_N
LAT = 32
EDGES = 1600000
RIS = 0.2

NC = 2
NS = 16
HALF = NODES // NC
DUMP = 1280
ACC_ROWS = HALF + DUMP
ROWS_MAIN = 3128
ROWS_LAST = HALF - 15 * ROWS_MAIN

EPAD = 1638400
ES = EPAD // NS
BATCH = 1024
SUB = 128
NBATCH = ES // BATCH
NSUB = BATCH // SUB


def _spmm_body(dst_hbm, src_hbm, val_hbm, table_hbm, init_hbm, out_hbm,
               acc, dstb, srcb, valb,
               src_sc0, src_sc1, idx_sc0, idx_sc1, rows0, rows1,
               semg0, semg1):
    c = lax.axis_index("c")
    s = lax.axis_index("s")
    lo = c * HALF
    r0 = s * ROWS_MAIN
    sbufs = (src_sc0, src_sc1)
    ibufs = (idx_sc0, idx_sc1)
    rbufs = (rows0, rows1)
    sems = (semg0, semg1)

    @pl.when(s < NS - 1)
    def _():
        pltpu.sync_copy(init_hbm.at[pl.ds(lo + r0, ROWS_MAIN)],
                        acc.at[pl.ds(r0, ROWS_MAIN)])

    @pl.when(s == NS - 1)
    def _():
        pltpu.sync_copy(init_hbm.at[pl.ds(lo + r0, ROWS_LAST)],
                        acc.at[pl.ds(r0, ROWS_LAST)])

    plsc.subcore_barrier()

    iota = lax.iota(jnp.int32, 16)
    e_base = s * ES

    def prep(k, sbuf, ibuf):
        for j8 in range(SUB // 16):
            o = k * SUB + j8 * 16
            d16 = dstb[pl.ds(o, 16)]
            rl = d16 - lo
            mask = (rl >= 0) & (rl < HALF)
            dump = HALF + ((k * (SUB // 16) + j8) % (DUMP // 16)) * 16 + iota
            ibuf[pl.ds(j8 * 16, 16)] = jnp.where(mask, rl, dump)
            sbuf[pl.ds(j8 * 16, 16)] = srcb[pl.ds(o, 16)]
            v16 = valb[pl.ds(o, 16)]
            valb[pl.ds(o, 16)] = jnp.where(mask, v16, 0.0)

    def mul_rows(k, rbuf):
        def mul(r, _):
            bv = plsc.load_gather(
                valb, [jnp.full((16,), k * SUB + r, jnp.int32)])
            rbuf[r, pl.ds(0, 16)] = rbuf[r, pl.ds(0, 16)] * bv
            rbuf[r, pl.ds(16, 16)] = rbuf[r, pl.ds(16, 16)] * bv
            return 0
        lax.fori_loop(0, SUB, mul, 0, unroll=4)

    def batch_body(b, _):
        e0 = e_base + b * BATCH
        pltpu.sync_copy(dst_hbm.at[pl.ds(e0, BATCH)], dstb)
        pltpu.sync_copy(src_hbm.at[pl.ds(e0, BATCH)], srcb)
        pltpu.sync_copy(val_hbm.at[pl.ds(e0, BATCH)], valb)

        prep(0, sbufs[0], ibufs[0])
        gathers = [pltpu.async_copy(table_hbm.at[sbufs[0]], rbufs[0], sems[0])]
        for k in range(NSUB):
            cur = k % 2
            nxt = 1 - cur
            if k + 1 < NSUB:
                prep(k + 1, sbufs[nxt], ibufs[nxt])
                gathers.append(pltpu.async_copy(
                    table_hbm.at[sbufs[nxt]], rbufs[nxt], sems[nxt]))
            gathers[k].wait()
            mul_rows(k, rbufs[cur])
            pltpu.sync_copy(rbufs[cur], acc.at[ibufs[cur]], add=True)
        return 0

    lax.fori_loop(0, NBATCH, batch_body, 0)
    plsc.subcore_barrier()

    @pl.when(s < NS - 1)
    def _():
        pltpu.sync_copy(acc.at[pl.ds(r0, ROWS_MAIN)],
                        out_hbm.at[pl.ds(lo + r0, ROWS_MAIN)])

    @pl.when(s == NS - 1)
    def _():
        pltpu.sync_copy(acc.at[pl.ds(r0, ROWS_LAST)],
                        out_hbm.at[pl.ds(lo + r0, ROWS_LAST)])


_spmm_call = pl.kernel(
    _spmm_body,
    out_type=jax.ShapeDtypeStruct((NODES, LAT), jnp.float32),
    mesh=plsc.VectorSubcoreMesh(core_axis_name="c", subcore_axis_name="s"),
    scratch_types=[
        pltpu.VMEM_SHARED((ACC_ROWS, LAT), jnp.float32),
        pltpu.VMEM((BATCH,), jnp.int32),
        pltpu.VMEM((BATCH,), jnp.int32),
        pltpu.VMEM((BATCH,), jnp.float32),
        pltpu.VMEM((SUB,), jnp.int32),
        pltpu.VMEM((SUB,), jnp.int32),
        pltpu.VMEM((SUB,), jnp.int32),
        pltpu.VMEM((SUB,), jnp.int32),
        pltpu.VMEM((SUB, LAT), jnp.float32),
        pltpu.VMEM((SUB, LAT), jnp.float32),
        pltpu.SemaphoreType.DMA,
        pltpu.SemaphoreType.DMA,
    ],
    compiler_params=pltpu.CompilerParams(needs_layout_passes=False,
                                         use_tc_tiling_on_sc=False),
)


def _spmm(dst, src, val, table, init):
    return _spmm_call(dst, src, val, table, init)


def _feat_kernel(emb_ref, w_ref, o_ref):
    x = jnp.dot(emb_ref[...], w_ref[...], preferred_element_type=jnp.float32)
    y = jnp.where(x >= 0, x, 0.2 * x)
    n = jnp.sqrt(jnp.sum(y * y, axis=1, keepdims=True))
    o_ref[...] = y / jnp.maximum(n, 1e-12)


def _feat_norm(emb, w):
    bm = 400
    f = emb.shape[1]
    return pl.pallas_call(
        _feat_kernel,
        grid=(ITEM_N // bm,),
        in_specs=[pl.BlockSpec((bm, f), lambda i: (i, 0)),
                  pl.BlockSpec((f, LAT), lambda i: (0, 0))],
        out_specs=pl.BlockSpec((bm, LAT), lambda i: (i, 0)),
        out_shape=jax.ShapeDtypeStruct((ITEM_N, LAT), jnp.float32),
    )(emb, w)


def _axpy_kernel(scale, a_ref, b_ref, o_ref):
    o_ref[...] = a_ref[...] + scale * b_ref[...]


def _axpy(a, b, scale):
    bm = 1000
    return pl.pallas_call(
        functools.partial(_axpy_kernel, scale),
        grid=(NODES // bm,),
        in_specs=[pl.BlockSpec((bm, LAT), lambda i: (i, 0))] * 2,
        out_specs=pl.BlockSpec((bm, LAT), lambda i: (i, 0)),
        out_shape=jax.ShapeDtypeStruct((NODES, LAT), jnp.float32),
    )(a, b)


def kernel(adj_indices, adj_values, image_adj_indices, image_adj_values,
           text_adj_indices, text_adj_values, image_embedding, text_embedding,
           uEmbeds, iEmbeds, image_trans, text_trans):
    pad = EPAD - EDGES
    pad_idx = (jnp.arange(pad, dtype=jnp.int32) * 997) % NODES
    pad_val = jnp.zeros((pad,), jnp.float32)

    def split_edges(indices, values):
        dst = jnp.concatenate([indices[0], pad_idx])
        src = jnp.concatenate([indices[1], pad_idx])
        val = jnp.concatenate([values, pad_val])
        return dst, src, val

    a_dst, a_src, a_val = split_edges(adj_indices, adj_values)
    i_dst, i_src, i_val = split_edges(image_adj_indices, image_adj_values)
    t_dst, t_src, t_val = split_edges(text_adj_indices, text_adj_values)

    img_n = _feat_norm(image_embedding, image_trans)
    txt_n = _feat_norm(text_embedding, text_trans)
    base = jnp.concatenate([uEmbeds, iEmbeds], axis=0)
    x_img = jnp.concatenate([uEmbeds, img_n], axis=0)
    x_txt = jnp.concatenate([uEmbeds, txt_n], axis=0)
    zeros = jnp.zeros((NODES, LAT), jnp.float32)

    def branch(dstv, srcv, valv, x_in):
        a_m = _spmm(dstv, srcv, valv, base, zeros)
        y = _spmm(a_dst, a_src, a_val, x_in, zeros)
        p = _axpy(y, a_m, RIS)
        x2 = jnp.concatenate([y[:USER_N], iEmbeds], axis=0)
        z0 = _spmm(a_dst, a_src, a_val, x2, p)
        z1 = _spmm(a_dst, a_src, a_val, z0, zeros)
        q = _axpy(z0, z1, 1.0)
        return _spmm(a_dst, a_src, a_val, z1, q)

    f_img = branch(i_dst, i_src, i_val, x_img)
    f_txt = branch(t_dst, t_src, t_val, x_txt)

    embeds = jnp.concatenate([f_img, f_txt], axis=-1)
    return embeds[:USER_N], embeds[USER_N:]


# speedup vs baseline: 7.5166x; 1.3750x over previous
import functools

import jax
import jax.numpy as jnp
from jax import lax
from jax.experimental import pallas as pl
from jax.experimental.pallas import tpu as pltpu
from jax.experimental.pallas import tpu_sc as plsc

USER_N = 60000
ITEM_N = 40000
NODES = USER_N + ITEM_N
LAT = 32
EDGES = 1600000
RIS = 0.2

NC = 2
NS = 16
HALF = NODES // NC
DUMP = 1280
ACC_ROWS = HALF + DUMP
ROWS_MAIN = 3128
ROWS_LAST = HALF - 15 * ROWS_MAIN

EPAD = 1638400
ES = EPAD // NS
BATCH = 1024
SUB = 128
NBATCH = ES // BATCH
NSUB = BATCH // SUB


def _spmm_body(dst_hbm, src_hbm, val_hbm, table_hbm, init_hbm, out_hbm,
               acc, dstb, srcb, valb,
               src_sc0, src_sc1, idx_sc0, idx_sc1, rows0, rows1,
               semg0, semg1):
    c = lax.axis_index("c")
    s = lax.axis_index("s")
    lo = c * HALF
    r0 = s * ROWS_MAIN
    sbufs = (src_sc0, src_sc1)
    ibufs = (idx_sc0, idx_sc1)
    rbufs = (rows0, rows1)
    sems = (semg0, semg1)

    @pl.when(s < NS - 1)
    def _():
        pltpu.sync_copy(init_hbm.at[pl.ds(lo + r0, ROWS_MAIN)],
                        acc.at[pl.ds(r0, ROWS_MAIN)])

    @pl.when(s == NS - 1)
    def _():
        pltpu.sync_copy(init_hbm.at[pl.ds(lo + r0, ROWS_LAST)],
                        acc.at[pl.ds(r0, ROWS_LAST)])

    plsc.subcore_barrier()

    iota = lax.iota(jnp.int32, 16)
    e_base = s * ES

    def prep(k, sbuf, ibuf):
        for j8 in range(SUB // 16):
            o = k * SUB + j8 * 16
            d16 = dstb[pl.ds(o, 16)]
            rl = d16 - lo
            mask = (rl >= 0) & (rl < HALF)
            dump = HALF + ((k * (SUB // 16) + j8) % (DUMP // 16)) * 16 + iota
            ibuf[pl.ds(j8 * 16, 16)] = jnp.where(mask, rl, dump)
            sbuf[pl.ds(j8 * 16, 16)] = srcb[pl.ds(o, 16)]
            v16 = valb[pl.ds(o, 16)]
            valb[pl.ds(o, 16)] = jnp.where(mask, v16, 0.0)

    def mul_rows(k, rbuf):
        def mul16(g, _):
            v16 = valb[pl.ds(k * SUB + g * 16, 16)]
            for i in range(16):
                bv = jnp.take_along_axis(
                    v16, jnp.full((16,), i, jnp.int32), axis=0)
                r = g * 16 + i
                rbuf[r, pl.ds(0, 16)] = rbuf[r, pl.ds(0, 16)] * bv
                rbuf[r, pl.ds(16, 16)] = rbuf[r, pl.ds(16, 16)] * bv
            return 0
        lax.fori_loop(0, SUB // 16, mul16, 0)

    def batch_body(b, _):
        e0 = e_base + b * BATCH
        pltpu.sync_copy(dst_hbm.at[pl.ds(e0, BATCH)], dstb)
        pltpu.sync_copy(src_hbm.at[pl.ds(e0, BATCH)], srcb)
        pltpu.sync_copy(val_hbm.at[pl.ds(e0, BATCH)], valb)

        prep(0, sbufs[0], ibufs[0])
        gathers = [pltpu.async_copy(table_hbm.at[sbufs[0]], rbufs[0], sems[0])]
        for k in range(NSUB):
            cur = k % 2
            nxt = 1 - cur
            if k + 1 < NSUB:
                prep(k + 1, sbufs[nxt], ibufs[nxt])
                gathers.append(pltpu.async_copy(
                    table_hbm.at[sbufs[nxt]], rbufs[nxt], sems[nxt]))
            gathers[k].wait()
            mul_rows(k, rbufs[cur])
            pltpu.sync_copy(rbufs[cur], acc.at[ibufs[cur]], add=True)
        return 0

    lax.fori_loop(0, NBATCH, batch_body, 0)
    plsc.subcore_barrier()

    @pl.when(s < NS - 1)
    def _():
        pltpu.sync_copy(acc.at[pl.ds(r0, ROWS_MAIN)],
                        out_hbm.at[pl.ds(lo + r0, ROWS_MAIN)])

    @pl.when(s == NS - 1)
    def _():
        pltpu.sync_copy(acc.at[pl.ds(r0, ROWS_LAST)],
                        out_hbm.at[pl.ds(lo + r0, ROWS_LAST)])


_spmm_call = pl.kernel(
    _spmm_body,
    out_type=jax.ShapeDtypeStruct((NODES, LAT), jnp.float32),
    mesh=plsc.VectorSubcoreMesh(core_axis_name="c", subcore_axis_name="s"),
    scratch_types=[
        pltpu.VMEM_SHARED((ACC_ROWS, LAT), jnp.float32),
        pltpu.VMEM((BATCH,), jnp.int32),
        pltpu.VMEM((BATCH,), jnp.int32),
        pltpu.VMEM((BATCH,), jnp.float32),
        pltpu.VMEM((SUB,), jnp.int32),
        pltpu.VMEM((SUB,), jnp.int32),
        pltpu.VMEM((SUB,), jnp.int32),
        pltpu.VMEM((SUB,), jnp.int32),
        pltpu.VMEM((SUB, LAT), jnp.float32),
        pltpu.VMEM((SUB, LAT), jnp.float32),
        pltpu.SemaphoreType.DMA,
        pltpu.SemaphoreType.DMA,
    ],
    compiler_params=pltpu.CompilerParams(needs_layout_passes=False,
                                         use_tc_tiling_on_sc=False),
)


def _spmm(dst, src, val, table, init):
    return _spmm_call(dst, src, val, table, init)


def _feat_kernel(emb_ref, w_ref, o_ref):
    x = jnp.dot(emb_ref[...], w_ref[...], preferred_element_type=jnp.float32)
    y = jnp.where(x >= 0, x, 0.2 * x)
    n = jnp.sqrt(jnp.sum(y * y, axis=1, keepdims=True))
    o_ref[...] = y / jnp.maximum(n, 1e-12)


def _feat_norm(emb, w):
    bm = 400
    f = emb.shape[1]
    return pl.pallas_call(
        _feat_kernel,
        grid=(ITEM_N // bm,),
        in_specs=[pl.BlockSpec((bm, f), lambda i: (i, 0)),
                  pl.BlockSpec((f, LAT), lambda i: (0, 0))],
        out_specs=pl.BlockSpec((bm, LAT), lambda i: (i, 0)),
        out_shape=jax.ShapeDtypeStruct((ITEM_N, LAT), jnp.float32),
    )(emb, w)


def _axpy_kernel(scale, a_ref, b_ref, o_ref):
    o_ref[...] = a_ref[...] + scale * b_ref[...]


def _axpy(a, b, scale):
    bm = 1000
    return pl.pallas_call(
        functools.partial(_axpy_kernel, scale),
        grid=(NODES // bm,),
        in_specs=[pl.BlockSpec((bm, LAT), lambda i: (i, 0))] * 2,
        out_specs=pl.BlockSpec((bm, LAT), lambda i: (i, 0)),
        out_shape=jax.ShapeDtypeStruct((NODES, LAT), jnp.float32),
    )(a, b)


def kernel(adj_indices, adj_values, image_adj_indices, image_adj_values,
           text_adj_indices, text_adj_values, image_embedding, text_embedding,
           uEmbeds, iEmbeds, image_trans, text_trans):
    pad = EPAD - EDGES
    pad_idx = (jnp.arange(pad, dtype=jnp.int32) * 997) % NODES
    pad_val = jnp.zeros((pad,), jnp.float32)

    def split_edges(indices, values):
        dst = jnp.concatenate([indices[0], pad_idx])
        src = jnp.concatenate([indices[1], pad_idx])
        val = jnp.concatenate([values, pad_val])
        return dst, src, val

    a_dst, a_src, a_val = split_edges(adj_indices, adj_values)
    i_dst, i_src, i_val = split_edges(image_adj_indices, image_adj_values)
    t_dst, t_src, t_val = split_edges(text_adj_indices, text_adj_values)

    img_n = _feat_norm(image_embedding, image_trans)
    txt_n = _feat_norm(text_embedding, text_trans)
    base = jnp.concatenate([uEmbeds, iEmbeds], axis=0)
    x_img = jnp.concatenate([uEmbeds, img_n], axis=0)
    x_txt = jnp.concatenate([uEmbeds, txt_n], axis=0)
    zeros = jnp.zeros((NODES, LAT), jnp.float32)

    def branch(dstv, srcv, valv, x_in):
        a_m = _spmm(dstv, srcv, valv, base, zeros)
        y = _spmm(a_dst, a_src, a_val, x_in, zeros)
        p = _axpy(y, a_m, RIS)
        x2 = jnp.concatenate([y[:USER_N], iEmbeds], axis=0)
        z0 = _spmm(a_dst, a_src, a_val, x2, p)
        z1 = _spmm(a_dst, a_src, a_val, z0, zeros)
        q = _axpy(z0, z1, 1.0)
        return _spmm(a_dst, a_src, a_val, z1, q)

    f_img = branch(i_dst, i_src, i_val, x_img)
    f_txt = branch(t_dst, t_src, t_val, x_txt)

    embeds = jnp.concatenate([f_img, f_txt], axis=-1)
    return embeds[:USER_N], embeds[USER_N:]


# speedup vs baseline: 11.6290x; 1.5471x over previous
import functools

import jax
import jax.numpy as jnp
from jax import lax
from jax.experimental import pallas as pl
from jax.experimental.pallas import tpu as pltpu
from jax.experimental.pallas import tpu_sc as plsc

USER_N = 60000
ITEM_N = 40000
NODES = USER_N + ITEM_N
LAT = 32
EDGES = 1600000
RIS = 0.2

NC = 2
NS = 16
NW = NC * NS
HALF = NODES // NC
DUMP = 1280
ACC_ROWS = HALF + DUMP
ROWS_MAIN = 3128
ROWS_LAST = HALF - 15 * ROWS_MAIN

EPAD = 1638400
ES2 = EPAD // NW
BATCH = 1024
SUB = 128
NSUB = BATCH // SUB
NPB = ES2 // BATCH
NBLK_CAP = NPB + 1
CAP_E = NBLK_CAP * BATCH

_SC_PARAMS = pltpu.CompilerParams(needs_layout_passes=False,
                                  use_tc_tiling_on_sc=False)
_SC_MESH = plsc.VectorSubcoreMesh(core_axis_name="c", subcore_axis_name="s")


def _prep_body(dst_hbm, src_hbm, val_hbm,
               bsrc_hbm, bidx_hbm, bval_hbm, nblk_hbm,
               dstb, srcb, valb, s0, i0, v0, s1, i1, v1, cbuf, cnts):
    c = lax.axis_index("c")
    s = lax.axis_index("s")
    w = c * NS + s
    e_base = w * ES2
    iota = lax.iota(jnp.int32, 16)
    stage = ((s0, i0, v0), (s1, i1, v1))

    for t in range(4):
        cnts[t] = 0

    def batch_body(b, _):
        e0 = e_base + b * BATCH
        pltpu.sync_copy(dst_hbm.at[pl.ds(e0, BATCH)], dstb)
        pltpu.sync_copy(src_hbm.at[pl.ds(e0, BATCH)], srcb)
        pltpu.sync_copy(val_hbm.at[pl.ds(e0, BATCH)], valb)

        def group(j, _):
            d16 = dstb[pl.ds(j * 16, 16)]
            s16 = srcb[pl.ds(j * 16, 16)]
            v16 = valb[pl.ds(j * 16, 16)]
            for h in range(NC):
                sb, ib, vb = stage[h]
                rl = d16 - h * HALF
                mask = (rl >= 0) & (rl < HALF)
                mi = mask.astype(jnp.int32)
                cnt = cnts[2 * h]
                pos = cnt + plsc.cumsum(mi) - mi
                plsc.store_scatter(sb, [pos], s16, mask=mask)
                plsc.store_scatter(ib, [pos], rl, mask=mask)
                plsc.store_scatter(vb, [pos], v16, mask=mask)
                newcnt = cnt + jnp.sum(mi)
                cnts[2 * h] = newcnt

                @pl.when(newcnt >= BATCH)
                def _():
                    nb = cnts[2 * h + 1]
                    off = pl.multiple_of(nb * BATCH, BATCH)
                    pltpu.sync_copy(sb.at[pl.ds(0, BATCH)],
                                    bsrc_hbm.at[h, w, pl.ds(off, BATCH)])
                    pltpu.sync_copy(ib.at[pl.ds(0, BATCH)],
                                    bidx_hbm.at[h, w, pl.ds(off, BATCH)])
                    pltpu.sync_copy(vb.at[pl.ds(0, BATCH)],
                                    bval_hbm.at[h, w, pl.ds(off, BATCH)])
                    sb[pl.ds(0, 16)] = sb[pl.ds(BATCH, 16)]
                    ib[pl.ds(0, 16)] = ib[pl.ds(BATCH, 16)]
                    vb[pl.ds(0, 16)] = vb[pl.ds(BATCH, 16)]
                    cnts[2 * h] = newcnt - BATCH
                    cnts[2 * h + 1] = nb + 1
            return 0

        lax.fori_loop(0, BATCH // 16, group, 0)
        return 0

    lax.fori_loop(0, NPB, batch_body, 0)

    for h in range(NC):
        sb, ib, vb = stage[h]
        cnt = cnts[2 * h]

        def padgrp(j, _):
            base = j * 16
            pos = base + iota
            keep = pos < cnt
            sb[pl.ds(base, 16)] = jnp.where(keep, sb[pl.ds(base, 16)], pos)
            ib[pl.ds(base, 16)] = jnp.where(keep, ib[pl.ds(base, 16)],
                                            HALF + pos)
            vb[pl.ds(base, 16)] = jnp.where(keep, vb[pl.ds(base, 16)], 0.0)
            return 0

        lax.fori_loop(0, BATCH // 16, padgrp, 0)
        nb = cnts[2 * h + 1]
        off = pl.multiple_of(nb * BATCH, BATCH)
        pltpu.sync_copy(sb.at[pl.ds(0, BATCH)],
                        bsrc_hbm.at[h, w, pl.ds(off, BATCH)])
        pltpu.sync_copy(ib.at[pl.ds(0, BATCH)],
                        bidx_hbm.at[h, w, pl.ds(off, BATCH)])
        pltpu.sync_copy(vb.at[pl.ds(0, BATCH)],
                        bval_hbm.at[h, w, pl.ds(off, BATCH)])
        cnts[2 * h + 1] = nb + 1

    nb0 = cnts[1]
    nb1 = cnts[3]
    cbuf[pl.ds(0, 16)] = jnp.where(
        iota == 0, nb0, jnp.where(iota == 1, nb1, 0))
    pltpu.sync_copy(cbuf, nblk_hbm.at[w])


_prep_call = pl.kernel(
    _prep_body,
    out_type=(
        jax.ShapeDtypeStruct((NC, NW, CAP_E), jnp.int32),
        jax.ShapeDtypeStruct((NC, NW, CAP_E), jnp.int32),
        jax.ShapeDtypeStruct((NC, NW, CAP_E), jnp.float32),
        jax.ShapeDtypeStruct((NW, 16), jnp.int32),
    ),
    mesh=_SC_MESH,
    scratch_types=[
        pltpu.VMEM((BATCH,), jnp.int32),
        pltpu.VMEM((BATCH,), jnp.int32),
        pltpu.VMEM((BATCH,), jnp.float32),
        pltpu.VMEM((BATCH + 16,), jnp.int32),
        pltpu.VMEM((BATCH + 16,), jnp.int32),
        pltpu.VMEM((BATCH + 16,), jnp.float32),
        pltpu.VMEM((BATCH + 16,), jnp.int32),
        pltpu.VMEM((BATCH + 16,), jnp.int32),
        pltpu.VMEM((BATCH + 16,), jnp.float32),
        pltpu.VMEM((16,), jnp.int32),
        pltpu.SMEM((8,), jnp.int32),
    ],
    compiler_params=_SC_PARAMS,
)


def _spmm_body(bsrc_hbm, bidx_hbm, bval_hbm, nblk_hbm, table_hbm,
               init_hbm, out_hbm,
               acc, nbuf, srcb, idxb, valb,
               sbuf0, sbuf1, ibuf0, ibuf1, rows0, rows1, semg0, semg1):
    c = lax.axis_index("c")
    s = lax.axis_index("s")
    lo = c * HALF
    r0 = s * ROWS_MAIN
    sbufs = (sbuf0, sbuf1)
    ibufs = (ibuf0, ibuf1)
    rbufs = (rows0, rows1)
    sems = (semg0, semg1)

    @pl.when(s < NS - 1)
    def _():
        pltpu.sync_copy(init_hbm.at[pl.ds(lo + r0, ROWS_MAIN)],
                        acc.at[pl.ds(r0, ROWS_MAIN)])

    @pl.when(s == NS - 1)
    def _():
        pltpu.sync_copy(init_hbm.at[pl.ds(lo + r0, ROWS_LAST)],
                        acc.at[pl.ds(r0, ROWS_LAST)])

    plsc.subcore_barrier()

    iota = lax.iota(jnp.int32, 16)

    def stage(k, sbuf, ibuf):
        for j8 in range(SUB // 16):
            o = k * SUB + j8 * 16
            sbuf[pl.ds(j8 * 16, 16)] = srcb[pl.ds(o, 16)]
            ibuf[pl.ds(j8 * 16, 16)] = idxb[pl.ds(o, 16)]

    def mul_rows(k, rbuf):
        def mul16(g, _):
            v16 = valb[pl.ds(k * SUB + g * 16, 16)]
            for i in range(16):
                bv = jnp.take_along_axis(
                    v16, jnp.full((16,), i, jnp.int32), axis=0)
                r = g * 16 + i
                rbuf[r, pl.ds(0, 16)] = rbuf[r, pl.ds(0, 16)] * bv
                rbuf[r, pl.ds(16, 16)] = rbuf[r, pl.ds(16, 16)] * bv
            return 0
        lax.fori_loop(0, SUB // 16, mul16, 0)

    for u in range(2):
        wsrc = 2 * s + u
        pltpu.sync_copy(nblk_hbm.at[wsrc], nbuf)
        nb16 = nbuf[pl.ds(0, 16)]
        nb = jnp.sum(jnp.where(iota == c, nb16, 0))

        def block_body(b, _):
            off = pl.multiple_of(b * BATCH, BATCH)
            pltpu.sync_copy(bsrc_hbm.at[c, wsrc, pl.ds(off, BATCH)], srcb)
            pltpu.sync_copy(bidx_hbm.at[c, wsrc, pl.ds(off, BATCH)], idxb)
            pltpu.sync_copy(bval_hbm.at[c, wsrc, pl.ds(off, BATCH)], valb)

            stage(0, sbufs[0], ibufs[0])
            gathers = [pltpu.async_copy(
                table_hbm.at[sbufs[0]], rbufs[0], sems[0])]
            for k in range(NSUB):
                cur = k % 2
                nxt = 1 - cur
                if k + 1 < NSUB:
                    stage(k + 1, sbufs[nxt], ibufs[nxt])
                    gathers.append(pltpu.async_copy(
                        table_hbm.at[sbufs[nxt]], rbufs[nxt], sems[nxt]))
                gathers[k].wait()
                mul_rows(k, rbufs[cur])
                pltpu.sync_copy(rbufs[cur], acc.at[ibufs[cur]], add=True)
            return 0

        lax.fori_loop(0, nb, block_body, 0)

    plsc.subcore_barrier()

    @pl.when(s < NS - 1)
    def _():
        pltpu.sync_copy(acc.at[pl.ds(r0, ROWS_MAIN)],
                        out_hbm.at[pl.ds(lo + r0, ROWS_MAIN)])

    @pl.when(s == NS - 1)
    def _():
        pltpu.sync_copy(acc.at[pl.ds(r0, ROWS_LAST)],
                        out_hbm.at[pl.ds(lo + r0, ROWS_LAST)])


_spmm_call = pl.kernel(
    _spmm_body,
    out_type=jax.ShapeDtypeStruct((NODES, LAT), jnp.float32),
    mesh=_SC_MESH,
    scratch_types=[
        pltpu.VMEM_SHARED((ACC_ROWS, LAT), jnp.float32),
        pltpu.VMEM((16,), jnp.int32),
        pltpu.VMEM((BATCH,), jnp.int32),
        pltpu.VMEM((BATCH,), jnp.int32),
        pltpu.VMEM((BATCH,), jnp.float32),
        pltpu.VMEM((SUB,), jnp.int32),
        pltpu.VMEM((SUB,), jnp.int32),
        pltpu.VMEM((SUB,), jnp.int32),
        pltpu.VMEM((SUB,), jnp.int32),
        pltpu.VMEM((SUB, LAT), jnp.float32),
        pltpu.VMEM((SUB, LAT), jnp.float32),
        pltpu.SemaphoreType.DMA,
        pltpu.SemaphoreType.DMA,
    ],
    compiler_params=_SC_PARAMS,
)


def _feat_kernel(emb_ref, w_ref, o_ref):
    x = jnp.dot(emb_ref[...], w_ref[...], preferred_element_type=jnp.float32)
    y = jnp.where(x >= 0, x, 0.2 * x)
    n = jnp.sqrt(jnp.sum(y * y, axis=1, keepdims=True))
    o_ref[...] = y / jnp.maximum(n, 1e-12)


def _feat_norm(emb, w):
    bm = 400
    f = emb.shape[1]
    return pl.pallas_call(
        _feat_kernel,
        grid=(ITEM_N // bm,),
        in_specs=[pl.BlockSpec((bm, f), lambda i: (i, 0)),
                  pl.BlockSpec((f, LAT), lambda i: (0, 0))],
        out_specs=pl.BlockSpec((bm, LAT), lambda i: (i, 0)),
        out_shape=jax.ShapeDtypeStruct((ITEM_N, LAT), jnp.float32),
    )(emb, w)


def _axpy_kernel(scale, a_ref, b_ref, o_ref):
    o_ref[...] = a_ref[...] + scale * b_ref[...]


def _axpy(a, b, scale):
    bm = 1000
    return pl.pallas_call(
        functools.partial(_axpy_kernel, scale),
        grid=(NODES // bm,),
        in_specs=[pl.BlockSpec((bm, LAT), lambda i: (i, 0))] * 2,
        out_specs=pl.BlockSpec((bm, LAT), lambda i: (i, 0)),
        out_shape=jax.ShapeDtypeStruct((NODES, LAT), jnp.float32),
    )(a, b)


def kernel(adj_indices, adj_values, image_adj_indices, image_adj_values,
           text_adj_indices, text_adj_values, image_embedding, text_embedding,
           uEmbeds, iEmbeds, image_trans, text_trans):
    pad = EPAD - EDGES
    pad_idx = (jnp.arange(pad, dtype=jnp.int32) * 997) % NODES
    pad_val = jnp.zeros((pad,), jnp.float32)

    def prep(indices, values):
        dst = jnp.concatenate([indices[0], pad_idx])
        src = jnp.concatenate([indices[1], pad_idx])
        val = jnp.concatenate([values, pad_val])
        return _prep_call(dst, src, val)

    adj_b = prep(adj_indices, adj_values)
    img_b = prep(image_adj_indices, image_adj_values)
    txt_b = prep(text_adj_indices, text_adj_values)

    def spmm(buckets, table, init):
        return _spmm_call(*buckets, table, init)

    img_n = _feat_norm(image_embedding, image_trans)
    txt_n = _feat_norm(text_embedding, text_trans)
    base = jnp.concatenate([uEmbeds, iEmbeds], axis=0)
    x_img = jnp.concatenate([uEmbeds, img_n], axis=0)
    x_txt = jnp.concatenate([uEmbeds, txt_n], axis=0)
    zeros = jnp.zeros((NODES, LAT), jnp.float32)

    def branch(mod_b, x_in):
        a_m = spmm(mod_b, base, zeros)
        y = spmm(adj_b, x_in, zeros)
        p = _axpy(y, a_m, RIS)
        x2 = jnp.concatenate([y[:USER_N], iEmbeds], axis=0)
        z0 = spmm(adj_b, x2, p)
        z1 = spmm(adj_b, z0, zeros)
        q = _axpy(z0, z1, 1.0)
        return spmm(adj_b, z1, q)

    f_img = branch(img_b, x_img)
    f_txt = branch(txt_b, x_txt)

    embeds = jnp.concatenate([f_img, f_txt], axis=-1)
    return embeds[:USER_N], embeds[USER_N:]
